# Initial kernel scaffold; baseline (speedup 1.0000x reference)
import functools

import jax
import jax.numpy as jnp
from jax import lax
from jax.experimental import pallas as pl
from jax.experimental.pallas import tpu as pltpu
from jax.experimental.pallas import tpu_sc as plsc

N = 100000
E = 3200000
DE = 16
DH = 128

NC = 2
NS = 16
NW = NC * NS

WIN = 1024
IDX_ROWS = WIN // 128
NWIN = E // WIN
FULL_ROUNDS = NWIN // NW
EXTRA = NWIN - FULL_ROUNDS * NW
ROWS_PER_SUB = N // NS


def _sc_segment_sum(edges, recv2d, zeros):
    mesh = plsc.VectorSubcoreMesh(core_axis_name="c", subcore_axis_name="s")

    @functools.partial(
        pl.kernel,
        mesh=mesh,
        out_type=jax.ShapeDtypeStruct((NC, N, DE), jnp.float32),
        scratch_types=[
            pltpu.VMEM((IDX_ROWS, 128), jnp.int32),
            pltpu.VMEM((WIN, DE), jnp.float32),
            pltpu.VMEM_SHARED((N, DE), jnp.float32),
        ],
    )
    def k(edges_hbm, recv_hbm, zeros_hbm, out_hbm, idx_v, rows_v, acc_sh):
        c = lax.axis_index("c")
        s = lax.axis_index("s")
        w = s * NC + c

        pltpu.sync_copy(
            zeros_hbm,
            acc_sh.at[pl.ds(s * ROWS_PER_SUB, ROWS_PER_SUB)],
        )
        plsc.subcore_barrier()

        def do_window(j):
            pltpu.sync_copy(recv_hbm.at[pl.ds(j * IDX_ROWS, IDX_ROWS)], idx_v)
            pltpu.sync_copy(edges_hbm.at[pl.ds(j * WIN, WIN)], rows_v)
            for jj in range(IDX_ROWS):
                pltpu.sync_copy(
                    rows_v.at[pl.ds(jj * 128, 128)],
                    acc_sh.at[idx_v.at[jj]],
                    add=True,
                )

        @pl.loop(0, FULL_ROUNDS)
        def _(i):
            do_window(i * NW + w)

        @pl.when(w < EXTRA)
        def _():
            do_window(FULL_ROUNDS * NW + w)

        plsc.subcore_barrier()
        pltpu.sync_copy(
            acc_sh.at[pl.ds(s * ROWS_PER_SUB, ROWS_PER_SUB)],
            out_hbm.at[c, pl.ds(s * ROWS_PER_SUB, ROWS_PER_SUB)],
        )

    return k(edges, recv2d, zeros)


_BLK = 2000


def _gru_body(p_ref, nodes_ref, wih_ref, whh_ref, b_ref, bn_ref, out_ref):
    aggr = p_ref[0] + p_ref[1]
    h = nodes_ref[...]
    ig = lax.dot_general(
        aggr, wih_ref[...], (((1,), (1,)), ((), ())),
        preferred_element_type=jnp.float32,
    ) + b_ref[...]
    hg = lax.dot_general(
        h, whh_ref[...], (((1,), (1,)), ((), ())),
        preferred_element_type=jnp.float32,
    )
    ir, iz, inew = ig[:, :DH], ig[:, DH:2 * DH], ig[:, 2 * DH:]
    hr, hz, hn = hg[:, :DH], hg[:, DH:2 * DH], hg[:, 2 * DH:]
    reset = jax.nn.sigmoid(ir + hr)
    inp = jax.nn.sigmoid(iz + hz)
    new = jnp.tanh(inew + reset * (hn + bn_ref[...]))
    out_ref[...] = new + inp * (h - new)


def _tc_gru(partials, nodes, W_ih, W_hh, b2, bn2):
    grid = (N // _BLK,)
    return pl.pallas_call(
        _gru_body,
        grid=grid,
        in_specs=[
            pl.BlockSpec((NC, _BLK, DE), lambda i: (0, i, 0)),
            pl.BlockSpec((_BLK, DH), lambda i: (i, 0)),
            pl.BlockSpec((3 * DH, DE), lambda i: (0, 0)),
            pl.BlockSpec((3 * DH, DH), lambda i: (0, 0)),
            pl.BlockSpec((1, 3 * DH), lambda i: (0, 0)),
            pl.BlockSpec((1, DH), lambda i: (0, 0)),
        ],
        out_specs=pl.BlockSpec((_BLK, DH), lambda i: (i, 0)),
        out_shape=jax.ShapeDtypeStruct((N, DH), jnp.float32),
    )(partials, nodes, W_ih, W_hh, b2, bn2)


def kernel(nodes, edges, receivers, senders, W_ih, W_hh, b, b_n):
    del senders
    recv2d = receivers.reshape(E // 128, 128)
    zeros = jnp.zeros((ROWS_PER_SUB, DE), jnp.float32)
    partials = _sc_segment_sum(edges, recv2d, zeros)
    return _tc_gru(
        partials, nodes, W_ih, W_hh,
        b.reshape(1, 3 * DH), b_n.reshape(1, DH),
    )


# speedup vs baseline: 5.8591x; 5.8591x over previous
import functools

import jax
import jax.numpy as jnp
from jax import lax
from jax.experimental import pallas as pl
from jax.experimental.pallas import tpu as pltpu
from jax.experimental.pallas import tpu_sc as plsc

N = 100000
E = 3200000
DE = 16
DH = 128

NC = 2
NS = 16
NW = NC * NS

WIN = 1024
IDX_ROWS = WIN // 128
NWIN = E // WIN
FULL_ROUNDS = NWIN // NW
EXTRA = NWIN - FULL_ROUNDS * NW
ROWS_PER_SUB = N // NS


def _sc_segment_sum(edges, recv2d, zeros):
    mesh = plsc.VectorSubcoreMesh(core_axis_name="c", subcore_axis_name="s")

    @functools.partial(
        pl.kernel,
        mesh=mesh,
        out_type=jax.ShapeDtypeStruct((NC, NS, ROWS_PER_SUB, DE), jnp.float32),
        scratch_types=[
            pltpu.VMEM((IDX_ROWS, 128), jnp.int32),
            pltpu.VMEM((WIN, DE), jnp.float32),
            pltpu.VMEM_SHARED((N, DE), jnp.float32),
        ],
        compiler_params=pltpu.CompilerParams(use_tc_tiling_on_sc=False),
    )
    def k(edges_hbm, recv_hbm, zeros_hbm, out_hbm, idx_v, rows_v, acc_sh):
        c = lax.axis_index("c")
        s = lax.axis_index("s")
        w = s * NC + c

        pltpu.sync_copy(
            zeros_hbm,
            acc_sh.at[pl.ds(s * ROWS_PER_SUB, ROWS_PER_SUB)],
        )
        plsc.subcore_barrier()

        def do_window(j):
            pltpu.sync_copy(recv_hbm.at[pl.ds(j * IDX_ROWS, IDX_ROWS)], idx_v)
            pltpu.sync_copy(edges_hbm.at[pl.ds(j * WIN, WIN)], rows_v)
            for jj in range(IDX_ROWS):
                pltpu.sync_copy(
                    rows_v.at[pl.ds(jj * 128, 128)],
                    acc_sh.at[idx_v.at[jj]],
                    add=True,
                )

        @pl.loop(0, FULL_ROUNDS)
        def _(i):
            do_window(i * NW + w)

        @pl.when(w < EXTRA)
        def _():
            do_window(FULL_ROUNDS * NW + w)

        plsc.subcore_barrier()
        pltpu.sync_copy(
            acc_sh.at[pl.ds(s * ROWS_PER_SUB, ROWS_PER_SUB)],
            out_hbm.at[c, s],
        )

    return k(edges, recv2d, zeros)


_BLK = 2000


def _gru_body(p_ref, nodes_ref, wih_ref, whh_ref, b_ref, bn_ref, out_ref):
    aggr = p_ref[0] + p_ref[1]
    h = nodes_ref[...]
    ig = lax.dot_general(
        aggr, wih_ref[...], (((1,), (1,)), ((), ())),
        preferred_element_type=jnp.float32,
    ) + b_ref[...]
    hg = lax.dot_general(
        h, whh_ref[...], (((1,), (1,)), ((), ())),
        preferred_element_type=jnp.float32,
    )
    ir, iz, inew = ig[:, :DH], ig[:, DH:2 * DH], ig[:, 2 * DH:]
    hr, hz, hn = hg[:, :DH], hg[:, DH:2 * DH], hg[:, 2 * DH:]
    reset = jax.nn.sigmoid(ir + hr)
    inp = jax.nn.sigmoid(iz + hz)
    new = jnp.tanh(inew + reset * (hn + bn_ref[...]))
    out_ref[...] = new + inp * (h - new)


def _tc_gru(partials, nodes, W_ih, W_hh, b2, bn2):
    grid = (N // _BLK,)
    return pl.pallas_call(
        _gru_body,
        grid=grid,
        in_specs=[
            pl.BlockSpec((NC, _BLK, DE), lambda i: (0, i, 0)),
            pl.BlockSpec((_BLK, DH), lambda i: (i, 0)),
            pl.BlockSpec((3 * DH, DE), lambda i: (0, 0)),
            pl.BlockSpec((3 * DH, DH), lambda i: (0, 0)),
            pl.BlockSpec((1, 3 * DH), lambda i: (0, 0)),
            pl.BlockSpec((1, DH), lambda i: (0, 0)),
        ],
        out_specs=pl.BlockSpec((_BLK, DH), lambda i: (i, 0)),
        out_shape=jax.ShapeDtypeStruct((N, DH), jnp.float32),
    )(partials, nodes, W_ih, W_hh, b2, bn2)


def kernel(nodes, edges, receivers, senders, W_ih, W_hh, b, b_n):
    del senders
    recv2d = receivers.reshape(E // 128, 128)
    zeros = jnp.zeros((ROWS_PER_SUB, DE), jnp.float32)
    partials = _sc_segment_sum(edges, recv2d, zeros)
    partials = partials.reshape(NC, N, DE)
    return _tc_gru(
        partials, nodes, W_ih, W_hh,
        b.reshape(1, 3 * DH), b_n.reshape(1, DH),
    )
